# baseline (speedup 1.0000x reference)
import functools

import jax
import jax.numpy as jnp
import numpy as np
from jax import lax
from jax.experimental import pallas as pl
from jax.experimental.pallas import tpu as pltpu
from jax.experimental.pallas import tpu_sc as plsc

HASH_MOD = 2**31 - 1
NUM_ATTRS = 100
R = 128
C = 128
N = R * C
NUM_WORKERS = 32
CHUNK = N // NUM_WORKERS
LANES = 16

_W = tuple(
    int(v)
    for v in (
        np.random.default_rng(1234)
        .integers(1, HASH_MOD, size=(NUM_ATTRS,), dtype=np.int64)
        .astype(np.int32)
        | 1
    )
)


def _row_iota():
    return lax.broadcasted_iota(jnp.int32, (R, C), 0)


def _lane_iota():
    return lax.broadcasted_iota(jnp.int32, (R, C), 1)


def _bit_zero(j):
    if j >= N:
        return jnp.full((R, C), True)
    if j < C:
        return (_lane_iota() & j) == 0
    return (_row_iota() & (j // C)) == 0


def _partner(x, j):
    if j < C:
        bit = _bit_zero(j)
        return jnp.where(bit, jnp.roll(x, -j, axis=1), jnp.roll(x, j, axis=1))
    jr = j // C
    bit = _bit_zero(j)
    return jnp.where(bit, jnp.roll(x, -jr, axis=0), jnp.roll(x, jr, axis=0))


def _cx_val(h, j, k):
    hp = _partner(h, j)
    tm = _bit_zero(j) == _bit_zero(k)
    keep = (h == hp) | ((h < hp) == tm)
    return jnp.where(keep, h, hp)


def _prefix_incl(x):
    lane = _lane_iota()
    for s in (1, 2, 4, 8, 16, 32, 64):
        x = x + jnp.where(lane >= s, jnp.roll(x, s, axis=1), 0)
    rowtot = jax.lax.broadcast_in_dim(x[:, C - 1], (R, C), (0,))
    row = _row_iota()
    for s in (1, 2, 4, 8, 16, 32, 64):
        rowtot = rowtot + jnp.where(row >= s, jnp.roll(rowtot, s, axis=0), 0)
    return x + jnp.where(row >= 1, jnp.roll(rowtot, 1, axis=0), 0)


def _tc_body(xt_ref, mask_ref, h_ref, s_ref, r_ref):
    acc = jnp.zeros((R, C), jnp.int32)
    bias = jnp.int32(0)
    for a in range(NUM_ATTRS):
        wm = jnp.where(mask_ref[0, a] != 0, jnp.int32(_W[a]), jnp.int32(0))
        acc = acc + xt_ref[a] * wm
        bias = bias + wm
    s = acc + bias
    h = s % HASH_MOD
    h = jnp.where(h < 0, h + HASH_MOD, h)
    h_ref[...] = h

    k = 2
    while k <= N:
        j = k // 2
        while j >= 1:
            h = _cx_val(h, j, k)
            j //= 2
        k *= 2
    s_ref[...] = h

    p1 = jnp.roll(h, 1, axis=1)
    prev = jnp.where(_lane_iota() == 0, jnp.roll(p1, 1, axis=0), p1)
    e0 = (_row_iota() == 0) & (_lane_iota() == 0)
    f = (e0 | (h != prev)).astype(jnp.int32)
    r_ref[...] = _prefix_incl(f) - 1


def _tc_stage(xt, mask_i32, interpret=False):
    return pl.pallas_call(
        _tc_body,
        out_shape=(
            jax.ShapeDtypeStruct((R, C), jnp.int32),
            jax.ShapeDtypeStruct((R, C), jnp.int32),
            jax.ShapeDtypeStruct((R, C), jnp.int32),
        ),
        in_specs=[
            pl.BlockSpec(memory_space=pltpu.VMEM),
            pl.BlockSpec(memory_space=pltpu.SMEM),
        ],
        out_specs=(
            pl.BlockSpec(memory_space=pltpu.VMEM),
            pl.BlockSpec(memory_space=pltpu.VMEM),
            pl.BlockSpec(memory_space=pltpu.VMEM),
        ),
        interpret=interpret,
    )(xt, mask_i32)


_BSEARCH_STEPS = (8192, 4096, 2048, 1024, 512, 256, 128, 64, 32, 16, 8, 4, 2, 1)


def _sc_lookup(h_flat, s_flat, r_flat):
    mesh = plsc.VectorSubcoreMesh(core_axis_name="c", subcore_axis_name="s")

    @functools.partial(
        pl.kernel,
        mesh=mesh,
        compiler_params=pltpu.CompilerParams(needs_layout_passes=False),
        out_type=jax.ShapeDtypeStruct((N,), jnp.int32),
        scratch_types=[
            pltpu.VMEM((CHUNK,), jnp.int32),
            pltpu.VMEM((N,), jnp.int32),
            pltpu.VMEM((N,), jnp.int32),
            pltpu.VMEM((CHUNK,), jnp.int32),
        ],
    )
    def k(h_hbm, s_hbm, r_hbm, out_hbm, h_v, s_v, r_v, o_v):
        wid = lax.axis_index("s") * 2 + lax.axis_index("c")
        base = wid * CHUNK
        pltpu.sync_copy(h_hbm.at[pl.ds(base, CHUNK)], h_v)
        pltpu.sync_copy(s_hbm, s_v)
        pltpu.sync_copy(r_hbm, r_v)

        def chunk(i, carry):
            hv = h_v[pl.ds(i * LANES, LANES)]
            lo = jnp.full((LANES,), -1, jnp.int32)
            for step in _BSEARCH_STEPS:
                mid = lo + step
                sv = plsc.load_gather(s_v, [mid])
                lo = jnp.where(sv < hv, mid, lo)
            o_v[pl.ds(i * LANES, LANES)] = plsc.load_gather(r_v, [lo + 1])
            return carry

        lax.fori_loop(0, CHUNK // LANES, chunk, 0)
        pltpu.sync_copy(o_v, out_hbm.at[pl.ds(base, CHUNK)])

    return k(h_flat, s_flat, r_flat)


def kernel(stacked_raw_attributes, blocks_mask):
    key = jax.random.key(42)
    k_idx, _k_branch, _k_split, _k_collide = jax.random.split(key, 4)
    n_blocks = blocks_mask.shape[0]
    random_index = jax.random.randint(k_idx, (), 0, n_blocks)
    chosen = blocks_mask[random_index]
    mask_i32 = chosen.astype(jnp.int32).reshape(1, NUM_ATTRS)
    xt = stacked_raw_attributes.T.reshape(NUM_ATTRS, R, C)
    h, s, r = _tc_stage(xt, mask_i32)
    return _sc_lookup(h.reshape(-1), s.reshape(-1), r.reshape(-1))


# speedup vs baseline: 1.0328x; 1.0328x over previous
import functools

import jax
import jax.numpy as jnp
import numpy as np
from jax import lax
from jax.experimental import pallas as pl
from jax.experimental.pallas import tpu as pltpu
from jax.experimental.pallas import tpu_sc as plsc

HASH_MOD = 2**31 - 1
NUM_ATTRS = 100
R = 128
C = 128
N = R * C
NUM_WORKERS = 32
CHUNK = N // NUM_WORKERS
LANES = 16

_W = tuple(
    int(v)
    for v in (
        np.random.default_rng(1234)
        .integers(1, HASH_MOD, size=(NUM_ATTRS,), dtype=np.int64)
        .astype(np.int32)
        | 1
    )
)


def _row_iota():
    return lax.broadcasted_iota(jnp.int32, (R, C), 0)


def _lane_iota():
    return lax.broadcasted_iota(jnp.int32, (R, C), 1)


def _bit_zero(j):
    if j >= N:
        return jnp.full((R, C), True)
    if j < C:
        return (_lane_iota() & j) == 0
    return (_row_iota() & (j // C)) == 0


def _partner(x, j):
    if j < C:
        bit = _bit_zero(j)
        return jnp.where(bit, jnp.roll(x, -j, axis=1), jnp.roll(x, j, axis=1))
    jr = j // C
    bit = _bit_zero(j)
    return jnp.where(bit, jnp.roll(x, -jr, axis=0), jnp.roll(x, jr, axis=0))


def _cx_val(h, j, k):
    hp = _partner(h, j)
    tm = _bit_zero(j) == _bit_zero(k)
    keep = (h == hp) | ((h < hp) == tm)
    return jnp.where(keep, h, hp)


def _prefix_incl(x):
    lane = _lane_iota()
    for s in (1, 2, 4, 8, 16, 32, 64):
        x = x + jnp.where(lane >= s, jnp.roll(x, s, axis=1), 0)
    rowtot = jax.lax.broadcast_in_dim(x[:, C - 1], (R, C), (0,))
    row = _row_iota()
    for s in (1, 2, 4, 8, 16, 32, 64):
        rowtot = rowtot + jnp.where(row >= s, jnp.roll(rowtot, s, axis=0), 0)
    return x + jnp.where(row >= 1, jnp.roll(rowtot, 1, axis=0), 0)


def _tc_body(xt_ref, mask_ref, h_ref, s_ref, r_ref):
    acc = jnp.zeros((R, C), jnp.int32)
    bias = jnp.int32(0)
    for a in range(NUM_ATTRS):
        wm = jnp.where(mask_ref[0, a] != 0, jnp.int32(_W[a]), jnp.int32(0))
        acc = acc + xt_ref[a] * wm
        bias = bias + wm
    s = acc + bias
    h = s % HASH_MOD
    h = jnp.where(h < 0, h + HASH_MOD, h)
    h_ref[...] = h

    k = 2
    while k <= N:
        j = k // 2
        while j >= 1:
            h = _cx_val(h, j, k)
            j //= 2
        k *= 2
    s_ref[...] = h

    p1 = jnp.roll(h, 1, axis=1)
    prev = jnp.where(_lane_iota() == 0, jnp.roll(p1, 1, axis=0), p1)
    e0 = (_row_iota() == 0) & (_lane_iota() == 0)
    f = (e0 | (h != prev)).astype(jnp.int32)
    r_ref[...] = _prefix_incl(f) - 1


def _tc_stage(xt, mask_i32, interpret=False):
    return pl.pallas_call(
        _tc_body,
        out_shape=(
            jax.ShapeDtypeStruct((R, C), jnp.int32),
            jax.ShapeDtypeStruct((R, C), jnp.int32),
            jax.ShapeDtypeStruct((R, C), jnp.int32),
        ),
        in_specs=[
            pl.BlockSpec(memory_space=pltpu.VMEM),
            pl.BlockSpec(memory_space=pltpu.SMEM),
        ],
        out_specs=(
            pl.BlockSpec(memory_space=pltpu.VMEM),
            pl.BlockSpec(memory_space=pltpu.VMEM),
            pl.BlockSpec(memory_space=pltpu.VMEM),
        ),
        interpret=interpret,
    )(xt, mask_i32)


_BSEARCH_STEPS = (8192, 4096, 2048, 1024, 512, 256, 128, 64, 32, 16, 8, 4, 2, 1)


def _sc_lookup(h_flat, s_flat, r_flat):
    mesh = plsc.VectorSubcoreMesh(core_axis_name="c", subcore_axis_name="s")

    @functools.partial(
        pl.kernel,
        mesh=mesh,
        compiler_params=pltpu.CompilerParams(needs_layout_passes=False),
        out_type=jax.ShapeDtypeStruct((N,), jnp.int32),
        scratch_types=[
            pltpu.VMEM((CHUNK,), jnp.int32),
            pltpu.VMEM((N,), jnp.int32),
            pltpu.VMEM((N,), jnp.int32),
            pltpu.VMEM((CHUNK,), jnp.int32),
        ],
    )
    def k(h_hbm, s_hbm, r_hbm, out_hbm, h_v, s_v, r_v, o_v):
        wid = lax.axis_index("s") * 2 + lax.axis_index("c")
        base = wid * CHUNK
        pltpu.sync_copy(h_hbm.at[pl.ds(base, CHUNK)], h_v)
        pltpu.sync_copy(s_hbm, s_v)
        pltpu.sync_copy(r_hbm, r_v)

        U = 4

        def chunk(i, carry):
            base_i = i * (LANES * U)
            hvs = [h_v[pl.ds(base_i + u * LANES, LANES)] for u in range(U)]
            los = [jnp.full((LANES,), -1, jnp.int32) for _ in range(U)]
            for step in _BSEARCH_STEPS:
                for u in range(U):
                    mid = los[u] + step
                    sv = plsc.load_gather(s_v, [mid])
                    los[u] = jnp.where(sv < hvs[u], mid, los[u])
            for u in range(U):
                o_v[pl.ds(base_i + u * LANES, LANES)] = plsc.load_gather(
                    r_v, [los[u] + 1]
                )
            return carry

        lax.fori_loop(0, CHUNK // (LANES * U), chunk, 0)
        pltpu.sync_copy(o_v, out_hbm.at[pl.ds(base, CHUNK)])

    return k(h_flat, s_flat, r_flat)


def kernel(stacked_raw_attributes, blocks_mask):
    key = jax.random.key(42)
    k_idx, _k_branch, _k_split, _k_collide = jax.random.split(key, 4)
    n_blocks = blocks_mask.shape[0]
    random_index = jax.random.randint(k_idx, (), 0, n_blocks)
    chosen = blocks_mask[random_index]
    mask_i32 = chosen.astype(jnp.int32).reshape(1, NUM_ATTRS)
    xt = stacked_raw_attributes.T.reshape(NUM_ATTRS, R, C)
    h, s, r = _tc_stage(xt, mask_i32)
    return _sc_lookup(h.reshape(-1), s.reshape(-1), r.reshape(-1))


# speedup vs baseline: 1.0364x; 1.0034x over previous
import functools

import jax
import jax.numpy as jnp
import numpy as np
from jax import lax
from jax.experimental import pallas as pl
from jax.experimental.pallas import tpu as pltpu
from jax.experimental.pallas import tpu_sc as plsc

HASH_MOD = 2**31 - 1
NUM_ATTRS = 100
R = 128
C = 128
N = R * C
NUM_WORKERS = 32
CHUNK = N // NUM_WORKERS
LANES = 16

_W = tuple(
    int(v)
    for v in (
        np.random.default_rng(1234)
        .integers(1, HASH_MOD, size=(NUM_ATTRS,), dtype=np.int64)
        .astype(np.int32)
        | 1
    )
)


def _row_iota():
    return lax.broadcasted_iota(jnp.int32, (R, C), 0)


def _lane_iota():
    return lax.broadcasted_iota(jnp.int32, (R, C), 1)


def _bit_zero(j):
    if j >= N:
        return jnp.full((R, C), True)
    if j < C:
        return (_lane_iota() & j) == 0
    return (_row_iota() & (j // C)) == 0


def _partner(x, j):
    if j < C:
        bit = _bit_zero(j)
        return jnp.where(bit, jnp.roll(x, -j, axis=1), jnp.roll(x, j, axis=1))
    jr = j // C
    bit = _bit_zero(j)
    return jnp.where(bit, jnp.roll(x, -jr, axis=0), jnp.roll(x, jr, axis=0))


def _cx_val(h, j, k):
    hp = _partner(h, j)
    tm = _bit_zero(j) == _bit_zero(k)
    keep = (h == hp) | ((h < hp) == tm)
    return jnp.where(keep, h, hp)


def _prefix_incl(x):
    lane = _lane_iota()
    for s in (1, 2, 4, 8, 16, 32, 64):
        x = x + jnp.where(lane >= s, jnp.roll(x, s, axis=1), 0)
    rowtot = jax.lax.broadcast_in_dim(x[:, C - 1], (R, C), (0,))
    row = _row_iota()
    for s in (1, 2, 4, 8, 16, 32, 64):
        rowtot = rowtot + jnp.where(row >= s, jnp.roll(rowtot, s, axis=0), 0)
    return x + jnp.where(row >= 1, jnp.roll(rowtot, 1, axis=0), 0)


def _tc_body(xt_ref, mask_ref, h_ref, s_ref, r_ref):
    acc = jnp.zeros((R, C), jnp.int32)
    bias = jnp.int32(0)
    for a in range(NUM_ATTRS):
        wm = jnp.where(mask_ref[0, a] != 0, jnp.int32(_W[a]), jnp.int32(0))
        acc = acc + xt_ref[a] * wm
        bias = bias + wm
    s = acc + bias
    h = s % HASH_MOD
    h = jnp.where(h < 0, h + HASH_MOD, h)
    h_ref[...] = h

    k = 2
    while k <= N:
        j = k // 2
        while j >= 1:
            h = _cx_val(h, j, k)
            j //= 2
        k *= 2
    s_ref[...] = h

    p1 = jnp.roll(h, 1, axis=1)
    prev = jnp.where(_lane_iota() == 0, jnp.roll(p1, 1, axis=0), p1)
    e0 = (_row_iota() == 0) & (_lane_iota() == 0)
    f = (e0 | (h != prev)).astype(jnp.int32)
    r_ref[...] = _prefix_incl(f) - 1


def _tc_stage(xt, mask_i32, interpret=False):
    return pl.pallas_call(
        _tc_body,
        out_shape=(
            jax.ShapeDtypeStruct((R, C), jnp.int32),
            jax.ShapeDtypeStruct((R, C), jnp.int32),
            jax.ShapeDtypeStruct((R, C), jnp.int32),
        ),
        in_specs=[
            pl.BlockSpec(memory_space=pltpu.VMEM),
            pl.BlockSpec(memory_space=pltpu.SMEM),
        ],
        out_specs=(
            pl.BlockSpec(memory_space=pltpu.VMEM),
            pl.BlockSpec(memory_space=pltpu.VMEM),
            pl.BlockSpec(memory_space=pltpu.VMEM),
        ),
        interpret=interpret,
    )(xt, mask_i32)


_BSEARCH_STEPS = (8192, 4096, 2048, 1024, 512, 256, 128, 64, 32, 16, 8, 4, 2, 1)


def _sc_lookup(h_flat, s_flat, r_flat):
    mesh = plsc.VectorSubcoreMesh(core_axis_name="c", subcore_axis_name="s")

    @functools.partial(
        pl.kernel,
        mesh=mesh,
        compiler_params=pltpu.CompilerParams(needs_layout_passes=False),
        out_type=jax.ShapeDtypeStruct((N,), jnp.int32),
        scratch_types=[
            pltpu.VMEM((CHUNK,), jnp.int32),
            pltpu.VMEM((N,), jnp.int32),
            pltpu.VMEM((N,), jnp.int32),
            pltpu.VMEM((CHUNK,), jnp.int32),
        ],
    )
    def k(h_hbm, s_hbm, r_hbm, out_hbm, h_v, s_v, r_v, o_v):
        wid = lax.axis_index("s") * 2 + lax.axis_index("c")
        base = wid * CHUNK
        pltpu.sync_copy(h_hbm.at[pl.ds(base, CHUNK)], h_v)
        pltpu.sync_copy(s_hbm, s_v)
        pltpu.sync_copy(r_hbm, r_v)

        U = 8

        def chunk(i, carry):
            base_i = i * (LANES * U)
            hvs = [h_v[pl.ds(base_i + u * LANES, LANES)] for u in range(U)]
            los = [jnp.full((LANES,), -1, jnp.int32) for _ in range(U)]
            for step in _BSEARCH_STEPS:
                for u in range(U):
                    mid = los[u] + step
                    sv = plsc.load_gather(s_v, [mid])
                    los[u] = jnp.where(sv < hvs[u], mid, los[u])
            for u in range(U):
                o_v[pl.ds(base_i + u * LANES, LANES)] = plsc.load_gather(
                    r_v, [los[u] + 1]
                )
            return carry

        lax.fori_loop(0, CHUNK // (LANES * U), chunk, 0)
        pltpu.sync_copy(o_v, out_hbm.at[pl.ds(base, CHUNK)])

    return k(h_flat, s_flat, r_flat)


def kernel(stacked_raw_attributes, blocks_mask):
    key = jax.random.key(42)
    k_idx, _k_branch, _k_split, _k_collide = jax.random.split(key, 4)
    n_blocks = blocks_mask.shape[0]
    random_index = jax.random.randint(k_idx, (), 0, n_blocks)
    chosen = blocks_mask[random_index]
    mask_i32 = chosen.astype(jnp.int32).reshape(1, NUM_ATTRS)
    xt = stacked_raw_attributes.T.reshape(NUM_ATTRS, R, C)
    h, s, r = _tc_stage(xt, mask_i32)
    return _sc_lookup(h.reshape(-1), s.reshape(-1), r.reshape(-1))
